# Initial kernel scaffold; baseline (speedup 1.0000x reference)
import jax
import jax.numpy as jnp
from jax.experimental import pallas as pl


def kernel(conv_data, emb_table, W0, W1, W_ih, W_hh, b_ih, b_hh, W_out, b_out):
    raise NotImplementedError("write your pallas kernel here")


# speedup vs baseline: 3.9319x; 3.9319x over previous
import functools

import jax
import jax.numpy as jnp
from jax import lax
from jax.experimental import pallas as pl
from jax.experimental.pallas import tpu as pltpu
from jax.experimental.pallas import tpu_sc as plsc

_NC = 2
_NS = 16
_NW = _NC * _NS


def _make_sc_gather(V, D, B, chunk):
    R = B // _NW
    n_chunk = R // chunk
    mesh = plsc.VectorSubcoreMesh(core_axis_name="c", subcore_axis_name="s")

    @functools.partial(
        pl.kernel,
        mesh=mesh,
        compiler_params=pltpu.CompilerParams(use_tc_tiling_on_sc=False),
        out_type=jax.ShapeDtypeStruct((B, D), jnp.float32),
        scratch_types=[
            pltpu.VMEM((n_chunk, chunk), jnp.int32),
            pltpu.VMEM((R, D), jnp.float32),
            pltpu.SemaphoreType.DMA,
        ],
    )
    def k(table_hbm, idx_hbm, out_hbm, idx_v, rows_v, sem):
        wid = lax.axis_index("s") * _NC + lax.axis_index("c")
        pltpu.sync_copy(idx_hbm.at[wid], idx_v)
        copies = [
            pltpu.async_copy(
                table_hbm.at[idx_v.at[j]],
                rows_v.at[pl.ds(j * chunk, chunk)],
                sem,
            )
            for j in range(n_chunk)
        ]
        for cp in copies:
            cp.wait()
        pltpu.sync_copy(rows_v, out_hbm.at[pl.ds(wid * R, R)])

    return k


def _sigmoid(x):
    return 1.0 / (1.0 + jnp.exp(-x))


def _lstm_scan(seq_ref, WihT_ref, WhhT_ref, bias_ref, xp_ref, h_ref, c_ref,
               T, B, H):
    xp_ref[...] = (
        jnp.dot(seq_ref[...], WihT_ref[...], preferred_element_type=jnp.float32)
        + bias_ref[...]
    )
    h_ref[...] = jnp.zeros((B, H), dtype=jnp.float32)
    c_ref[...] = jnp.zeros((B, H), dtype=jnp.float32)

    def step(t, _):
        gates = xp_ref[pl.ds(t * B, B), :] + jnp.dot(
            h_ref[...], WhhT_ref[...], preferred_element_type=jnp.float32
        )
        i = _sigmoid(gates[:, 0 * H:1 * H])
        f = _sigmoid(gates[:, 1 * H:2 * H])
        g = jnp.tanh(gates[:, 2 * H:3 * H])
        o = _sigmoid(gates[:, 3 * H:4 * H])
        c = f * c_ref[...] + i * g
        h_ref[...] = o * jnp.tanh(c)
        c_ref[...] = c
        return 0

    lax.fori_loop(0, T, step, 0)
    return h_ref[...]


def _tc1_body(seq_ref, WihT_ref, WhhT_ref, bias_ref, WoutT_ref, bout_ref,
              W0_ref, W1_ref, emb_ref, cur_ref, tgt_ref, tgtrow_ref,
              feat_ref, prehead_ref, xp_ref, h_ref, c_ref):
    B, H = h_ref.shape
    A = emb_ref.shape[0]
    T = seq_ref.shape[0] // B

    h = _lstm_scan(seq_ref, WihT_ref, WhhT_ref, bias_ref, xp_ref, h_ref,
                   c_ref, T, B, H)
    case = jnp.dot(h, WoutT_ref[...], preferred_element_type=jnp.float32)
    case = case + bout_ref[...]

    Gc = (lax.broadcasted_iota(jnp.int32, (B, A), 1) == cur_ref[...]
          ).astype(jnp.float32)
    Gt = (lax.broadcasted_iota(jnp.int32, (B, A), 1) == tgt_ref[...]
          ).astype(jnp.float32)
    GtT = (lax.broadcasted_iota(jnp.int32, (A, B), 0) == tgtrow_ref[...]
           ).astype(jnp.float32)

    def att(W_ref, feats):
        wf = jnp.dot(W_ref[...], feats, preferred_element_type=jnp.float32)
        h_emb = jnp.dot(Gc, feats, preferred_element_type=jnp.float32)
        t_emb = jnp.dot(Gt, feats, preferred_element_type=jnp.float32)
        diff = h_emb + case - t_emb
        d2 = jnp.sum(diff * diff, axis=1, keepdims=True)
        vals = jnp.exp(-jnp.sqrt(d2))
        norm = jnp.dot(GtT, vals, preferred_element_type=jnp.float32)
        normt = jnp.dot(Gt, norm, preferred_element_type=jnp.float32)
        scale = vals / (normt + 1e-12)
        wfc = jnp.dot(Gc, wf, preferred_element_type=jnp.float32)
        delta = jnp.dot(GtT, scale * wfc, preferred_element_type=jnp.float32)
        return jnp.maximum(wf + delta, 0.0)

    x1 = att(W0_ref, emb_ref[...])
    x2 = att(W1_ref, x1)
    feat_ref[...] = x2
    prehead_ref[...] = jnp.dot(Gt, x2, preferred_element_type=jnp.float32)


def _tc2_body(seq_ref, WihT_ref, WhhT_ref, bias_ref, WoutT_ref, bout_ref,
              prehead_ref, embT_ref, out_ref, xp_ref, h_ref, c_ref):
    B, H = h_ref.shape
    T = seq_ref.shape[0] // B

    h = _lstm_scan(seq_ref, WihT_ref, WhhT_ref, bias_ref, xp_ref, h_ref,
                   c_ref, T, B, H)
    pre_rel = jnp.dot(h, WoutT_ref[...], preferred_element_type=jnp.float32)
    pre_emb = prehead_ref[...] + pre_rel + bout_ref[...]

    embT = embT_ref[...]
    pn = jnp.sum(pre_emb * pre_emb, axis=1, keepdims=True)
    en = jnp.sum(embT * embT, axis=0, keepdims=True)
    cross = jnp.dot(pre_emb, embT, preferred_element_type=jnp.float32)
    d2 = jnp.maximum(pn + en - 2.0 * cross, 0.0)
    out_ref[...] = -jnp.sqrt(d2)


def kernel(conv_data, emb_table, W0, W1, W_ih, W_hh, b_ih, b_hh, W_out, b_out):
    A, F = emb_table.shape
    B, L = conv_data.shape
    H4 = W_ih.shape[0]
    H = H4 // 4
    T1, T2 = L - 1, L - 2

    conv = conv_data.astype(jnp.int32)
    idx1 = conv[:, :T1].T.reshape(T1 * B)
    idx2 = conv[:, :T2].T.reshape(T2 * B)
    cur = conv[:, L - 2].reshape(B, 1)
    tgt = conv[:, L - 1].reshape(B, 1)
    tgt_row = conv[:, L - 1].reshape(1, B)

    WihT = W_ih.T
    WhhT = W_hh.T
    bias = (b_ih + b_hh).reshape(1, H4)
    WoutT = W_out.T
    bout = b_out.reshape(1, F)
    embT = emb_table.T

    f32 = jnp.float32
    tc1 = pl.pallas_call(
        _tc1_body,
        out_shape=[
            jax.ShapeDtypeStruct((A, F), f32),
            jax.ShapeDtypeStruct((B, F), f32),
        ],
        scratch_shapes=[
            pltpu.VMEM((T1 * B, H4), f32),
            pltpu.VMEM((B, H), f32),
            pltpu.VMEM((B, H), f32),
        ],
    )
    tc2 = pl.pallas_call(
        _tc2_body,
        out_shape=jax.ShapeDtypeStruct((B, A), f32),
        scratch_shapes=[
            pltpu.VMEM((T2 * B, H4), f32),
            pltpu.VMEM((B, H), f32),
            pltpu.VMEM((B, H), f32),
        ],
    )

    chunk1 = 112
    chunk2 = 128
    seq1 = _make_sc_gather(A, F, T1 * B, chunk1)(
        emb_table, idx1.reshape(_NW, -1, chunk1))
    conv_feat, pre_head = tc1(seq1, WihT, WhhT, bias, WoutT, bout,
                              W0, W1, emb_table, cur, tgt, tgt_row)
    seq2 = _make_sc_gather(A, F, T2 * B, chunk2)(
        conv_feat, idx2.reshape(_NW, -1, chunk2))
    logits = tc2(seq2, WihT, WhhT, bias, WoutT, bout, pre_head, embT)
    return logits


# speedup vs baseline: 4.1385x; 1.0526x over previous
import functools

import jax
import jax.numpy as jnp
from jax import lax
from jax.experimental import pallas as pl
from jax.experimental.pallas import tpu as pltpu
from jax.experimental.pallas import tpu_sc as plsc

_NC = 2
_NS = 16
_NW = _NC * _NS


def _make_sc_gather(V, D, B, chunk):
    R = B // _NW
    n_chunk = R // chunk
    mesh = plsc.VectorSubcoreMesh(core_axis_name="c", subcore_axis_name="s")

    @functools.partial(
        pl.kernel,
        mesh=mesh,
        compiler_params=pltpu.CompilerParams(use_tc_tiling_on_sc=False),
        out_type=jax.ShapeDtypeStruct((B, D), jnp.float32),
        scratch_types=[
            pltpu.VMEM((n_chunk, chunk), jnp.int32),
            pltpu.VMEM((R, D), jnp.float32),
            pltpu.SemaphoreType.DMA,
        ],
    )
    def k(table_hbm, idx_hbm, out_hbm, idx_v, rows_v, sem):
        wid = lax.axis_index("s") * _NC + lax.axis_index("c")
        pltpu.sync_copy(idx_hbm.at[wid], idx_v)
        copies = [
            pltpu.async_copy(
                table_hbm.at[idx_v.at[j]],
                rows_v.at[pl.ds(j * chunk, chunk)],
                sem,
            )
            for j in range(n_chunk)
        ]
        for cp in copies:
            cp.wait()
        pltpu.sync_copy(rows_v, out_hbm.at[pl.ds(wid * R, R)])

    return k


def _sigmoid(x):
    return 1.0 / (1.0 + jnp.exp(-x))


def _lstm_weights(Wih_ref, Whh_ref, bih_ref, bhh_ref):
    Wcat = jnp.concatenate(
        [Wih_ref[...], Whh_ref[...]], axis=1).T.astype(jnp.bfloat16)
    bias = (bih_ref[...] + bhh_ref[...]).reshape(1, -1)
    return Wcat, bias


def _lstm_scan(seq3_ref, Wcat, bias, h_ref, c_ref, T, B, H):
    h_ref[...] = jnp.zeros((B, H), dtype=jnp.float32)
    c_ref[...] = jnp.zeros((B, H), dtype=jnp.float32)

    def step(t, _):
        x_t = seq3_ref[:, t, :]
        xh = jnp.concatenate([x_t, h_ref[...]], axis=1)
        gates = bias + jnp.dot(
            xh.astype(jnp.bfloat16), Wcat, preferred_element_type=jnp.float32
        )
        i = _sigmoid(gates[:, 0 * H:1 * H])
        f = _sigmoid(gates[:, 1 * H:2 * H])
        g = jnp.tanh(gates[:, 2 * H:3 * H])
        o = _sigmoid(gates[:, 3 * H:4 * H])
        c = f * c_ref[...] + i * g
        h_ref[...] = o * jnp.tanh(c)
        c_ref[...] = c
        return 0

    lax.fori_loop(0, T, step, 0, unroll=4)
    return h_ref[...]


def _tc1_body(seq3_ref, conv_ref, Wih_ref, Whh_ref, bih_ref, bhh_ref,
              Wout_ref, bout_ref, W0_ref, W1_ref, emb_ref,
              feat_ref, h_ref, c_ref):
    B, H = h_ref.shape
    A = emb_ref.shape[0]
    L = seq3_ref.shape[1]
    bf = jnp.bfloat16

    Wcat, bias = _lstm_weights(Wih_ref, Whh_ref, bih_ref, bhh_ref)
    h = _lstm_scan(seq3_ref, Wcat, bias, h_ref, c_ref, L - 1, B, H)
    case = jnp.dot(h.astype(bf), Wout_ref[...].T.astype(bf),
                   preferred_element_type=jnp.float32)
    case = case + bout_ref[...].reshape(1, -1)

    cur = conv_ref[:, L - 2:L - 1]
    tgt = conv_ref[:, L - 1:L]
    Gc = (lax.broadcasted_iota(jnp.int32, (B, A), 1) == cur).astype(bf)
    Gt = (lax.broadcasted_iota(jnp.int32, (B, A), 1) == tgt).astype(bf)
    Gd = Gc - Gt

    he1 = seq3_ref[:, L - 2, :]
    te1 = seq3_ref[:, L - 1, :]

    def att(W_ref, feats, diff):
        wf = jnp.dot(W_ref[...], feats, preferred_element_type=jnp.float32)
        d2 = jnp.sum(diff * diff, axis=1, keepdims=True)
        vals = jnp.exp(-jnp.sqrt(d2))
        wfc = jnp.dot(Gc, wf.astype(bf), preferred_element_type=jnp.float32)
        rhs = jnp.concatenate(
            [vals * wfc, jnp.broadcast_to(vals, wfc.shape)], axis=1
        ).astype(bf)
        dn = lax.dot_general(Gt, rhs, (((0,), (0,)), ((), ())),
                             preferred_element_type=jnp.float32)
        delta = dn[:, :wfc.shape[1]]
        norm = dn[:, wfc.shape[1]:wfc.shape[1] + 1]
        return jnp.maximum(wf + delta / (norm + 1e-12), 0.0)

    x1 = att(W0_ref, emb_ref[...], he1 + case - te1)
    diff2 = jnp.dot(Gd, x1.astype(bf),
                    preferred_element_type=jnp.float32) + case
    x2 = att(W1_ref, x1, diff2)
    feat_ref[...] = x2


def _tc2_body(seq3_ref, Wih_ref, Whh_ref, bih_ref, bhh_ref,
              Wout_ref, bout_ref, emb_ref, out_ref, h_ref, c_ref):
    B, H = h_ref.shape
    L = seq3_ref.shape[1]
    bf = jnp.bfloat16

    Wcat, bias = _lstm_weights(Wih_ref, Whh_ref, bih_ref, bhh_ref)
    h = _lstm_scan(seq3_ref, Wcat, bias, h_ref, c_ref, L - 2, B, H)
    pre_rel = jnp.dot(h.astype(bf), Wout_ref[...].T.astype(bf),
                      preferred_element_type=jnp.float32)
    pre_head = seq3_ref[:, L - 1, :]
    pre_emb = pre_head + pre_rel + bout_ref[...].reshape(1, -1)

    emb = emb_ref[...]
    pn = jnp.sum(pre_emb * pre_emb, axis=1, keepdims=True)
    en = jnp.sum(emb * emb, axis=1, keepdims=True)
    lhs = jnp.concatenate(
        [-2.0 * pre_emb, jnp.ones((B, 1), jnp.float32)], axis=1).astype(bf)
    rhsm = jnp.concatenate([emb, en], axis=1).astype(bf)
    d2 = pn + lax.dot_general(lhs, rhsm, (((1,), (1,)), ((), ())),
                              preferred_element_type=jnp.float32)
    out_ref[...] = -jnp.sqrt(jnp.maximum(d2, 0.0))


def kernel(conv_data, emb_table, W0, W1, W_ih, W_hh, b_ih, b_hh, W_out, b_out):
    A, F = emb_table.shape
    B, L = conv_data.shape
    H = W_hh.shape[1]

    conv = conv_data.astype(jnp.int32)
    chunk = 80
    idx3 = conv.reshape(_NW, -1, chunk)

    f32 = jnp.float32
    tc1 = pl.pallas_call(
        _tc1_body,
        out_shape=jax.ShapeDtypeStruct((A, F), f32),
        scratch_shapes=[
            pltpu.VMEM((B, H), f32),
            pltpu.VMEM((B, H), f32),
        ],
    )
    tc2 = pl.pallas_call(
        _tc2_body,
        out_shape=jax.ShapeDtypeStruct((B, A), f32),
        scratch_shapes=[
            pltpu.VMEM((B, H), f32),
            pltpu.VMEM((B, H), f32),
        ],
    )

    gather = _make_sc_gather(A, F, B * L, chunk)
    seq1 = gather(emb_table, idx3).reshape(B, L, F)
    conv_feat = tc1(seq1, conv, W_ih, W_hh, b_ih, b_hh, W_out, b_out,
                    W0, W1, emb_table)
    seq2 = gather(conv_feat, idx3).reshape(B, L, F)
    logits = tc2(seq2, W_ih, W_hh, b_ih, b_hh, W_out, b_out, emb_table)
    return logits
